# baseline (speedup 1.0000x reference)
import jax
import jax.numpy as jnp
from jax import lax
from jax.experimental import pallas as pl
from jax.experimental.pallas import tpu as pltpu
from jax.experimental.pallas import tpu_sc as plsc

_E = 8
_EMB = 768
_B = 16
_H = 256
_W = 320
_P = 16
_NP = (_H // _P) * (_W // _P)
_PATCH = 3 * _P * _P
_EPS = 2.220446049250313e-16


def _im2col_logits_body(x_ref, rv_ref, rb_ref, wg_ref, pat_ref, logits_t_ref,
                        acc_ref):
    b = pl.program_id(0)
    xb = x_ref[0]
    rows = []
    for gh in range(_H // _P):
        slab = xb[:, gh * _P:(gh + 1) * _P, :].reshape(3 * _P, _W)
        t = slab.T.reshape(_W // _P, _P, 3 * _P)
        rows.append(t.transpose(0, 2, 1).reshape(_W // _P, _PATCH))
    pat = jnp.concatenate(rows, axis=0)
    pat_ref[...] = pat
    feat = jnp.dot(pat, rv_ref[...],
                   preferred_element_type=jnp.float32) + rb_ref[0, 0]
    row = lax.dot_general(feat, wg_ref[...], (((0,), (0,)), ((), ())),
                          preferred_element_type=jnp.float32)
    acc_ref[pl.ds(b, 1), :] = row
    @pl.when(b == _B - 1)
    def _():
        logits_t_ref[...] = acc_ref[...].T


def _lane_gather(x, idx):
    return lax.gather(
        x, idx[:, None],
        lax.GatherDimensionNumbers(offset_dims=(), collapsed_slice_dims=(0,),
                                   start_index_map=(0,)),
        slice_sizes=(1,),
        mode=lax.GatherScatterMode.PROMISE_IN_BOUNDS)


def _bcast_lane(x, j):
    return _lane_gather(x, jnp.full((16,), j, jnp.int32))


def _route_body(logits_t_hbm, perm_out, eidx_out, loss_out, lbuf, perm_v,
                eidx_v, loss_v):
    @pl.when((lax.axis_index("c") == 0) & (lax.axis_index("s") == 0))
    def _():
        pltpu.sync_copy(logits_t_hbm, lbuf)
        lane = lax.iota(jnp.int32, 16)
        best = lbuf[0]
        bestidx = jnp.zeros((16,), jnp.int32)
        for e in range(1, _E):
            v = lbuf[e]
            gt = v > best
            best = jnp.where(gt, v, best)
            bestidx = jnp.where(gt, e, bestidx)
        counts_f = jnp.zeros((16,), jnp.float32)
        rank = jnp.zeros((16,), jnp.int32)
        for bp in range(_B):
            eb = _bcast_lane(bestidx, bp)
            counts_f = counts_f + jnp.where(lane == eb, 1.0, 0.0)
            before = (eb < bestidx) | ((eb == bestidx) & (bp < lane))
            rank = rank + jnp.where(before, 1, 0)
        perm = jnp.zeros((16,), jnp.int32)
        for bp in range(_B):
            rb = _bcast_lane(rank, bp)
            perm = jnp.where(lane == rb, bp, perm)
        eidx_sorted = _lane_gather(bestidx, perm)
        mean = _B / _E
        diff = jnp.where(lane < _E, counts_f - mean, 0.0)
        dsq = diff * diff
        tot = jnp.zeros((16,), jnp.float32)
        for e in range(_E):
            tot = tot + _bcast_lane(dsq, e)
        loss = tot * jnp.float32(2.0 / ((_E - 1) * (mean * mean + 1e-10)))
        perm_v[...] = perm
        eidx_v[...] = eidx_sorted
        loss_v[...] = loss
        pltpu.sync_copy(perm_v, perm_out)
        pltpu.sync_copy(eidx_v, eidx_out)
        pltpu.sync_copy(loss_v, loss_out)


def _dispatch_body(perm_ref, eidx_ref, p_ref, w_ref, b_ref, out_ref):
    w = w_ref[0].reshape(_PATCH, _EMB)
    y = jnp.dot(p_ref[...], w,
                preferred_element_type=jnp.float32) + b_ref[0]
    out_ref[...] = jnp.where(y == 0.0, jnp.float32(_EPS), y)


def kernel(x, router_w, router_b, expert_w, expert_b, w_gate, w_noise):
    rv = router_w.reshape(1, _PATCH).T
    w_e = expert_w.transpose(0, 2, 3, 4, 1)

    patches, logits_t = pl.pallas_call(
        _im2col_logits_body,
        grid=(_B,),
        in_specs=[
            pl.BlockSpec((1, 3, _H, _W), lambda b: (b, 0, 0, 0)),
            pl.BlockSpec((_PATCH, 1), lambda b: (0, 0)),
            pl.BlockSpec((1, 1), lambda b: (0, 0)),
            pl.BlockSpec((_NP, _E), lambda b: (0, 0)),
        ],
        out_specs=(
            pl.BlockSpec((_NP, _PATCH), lambda b: (b, 0)),
            pl.BlockSpec((_E, _B), lambda b: (0, 0)),
        ),
        out_shape=(
            jax.ShapeDtypeStruct((_B * _NP, _PATCH), jnp.float32),
            jax.ShapeDtypeStruct((_E, _B), jnp.float32),
        ),
        scratch_shapes=[pltpu.VMEM((_B, _E), jnp.float32)],
    )(x, rv, router_b.reshape(1, 1), w_gate)

    route = pl.kernel(
        _route_body,
        out_type=(
            jax.ShapeDtypeStruct((_B,), jnp.int32),
            jax.ShapeDtypeStruct((_B,), jnp.int32),
            jax.ShapeDtypeStruct((_B,), jnp.float32),
        ),
        mesh=plsc.VectorSubcoreMesh(core_axis_name="c", subcore_axis_name="s"),
        scratch_types=[
            pltpu.VMEM((_E, _B), jnp.float32),
            pltpu.VMEM((_B,), jnp.int32),
            pltpu.VMEM((_B,), jnp.int32),
            pltpu.VMEM((_B,), jnp.float32),
        ],
    )
    perm, eidx_sorted, loss_v = route(logits_t)

    out = pl.pallas_call(
        _dispatch_body,
        grid_spec=pltpu.PrefetchScalarGridSpec(
            num_scalar_prefetch=2,
            grid=(_B,),
            in_specs=[
                pl.BlockSpec((_NP, _PATCH),
                             lambda r, perm_ref, eidx_ref: (perm_ref[r], 0)),
                pl.BlockSpec((1, 3, _P, _P, _EMB),
                             lambda r, perm_ref, eidx_ref:
                             (eidx_ref[r], 0, 0, 0, 0)),
                pl.BlockSpec((1, 1, _EMB),
                             lambda r, perm_ref, eidx_ref: (eidx_ref[r], 0, 0)),
            ],
            out_specs=pl.BlockSpec((_NP, _EMB),
                                   lambda r, perm_ref, eidx_ref: (perm_ref[r], 0)),
        ),
        out_shape=jax.ShapeDtypeStruct((_B * _NP, _EMB), jnp.float32),
    )(perm, eidx_sorted, patches, w_e, expert_b.reshape(_E, 1, _EMB))
    return out.reshape(_B, _NP, _EMB), loss_v[0]


# speedup vs baseline: 1.0144x; 1.0144x over previous
import jax
import jax.numpy as jnp
from jax import lax
from jax.experimental import pallas as pl
from jax.experimental.pallas import tpu as pltpu
from jax.experimental.pallas import tpu_sc as plsc

_E = 8
_EMB = 768
_B = 16
_H = 256
_W = 320
_P = 16
_NP = (_H // _P) * (_W // _P)
_PATCH = 3 * _P * _P
_EPS = 2.220446049250313e-16


def _im2col_logits_body(x_ref, rv_ref, rb_ref, wg_ref, pat_ref, logits_t_ref,
                        acc_ref):
    b = pl.program_id(0)
    xb = x_ref[0].reshape(3, _H, _W)
    rows = []
    for gh in range(_H // _P):
        slab = xb[:, gh * _P:(gh + 1) * _P, :].reshape(3 * _P, _W)
        t = slab.T.reshape(_W // _P, _P, 3 * _P)
        rows.append(t.transpose(0, 2, 1).reshape(_W // _P, _PATCH))
    pat = jnp.concatenate(rows, axis=0)
    pat_ref[...] = pat
    feat = jnp.dot(pat, rv_ref[...],
                   preferred_element_type=jnp.float32) + rb_ref[0, 0]
    row = lax.dot_general(feat, wg_ref[...], (((0,), (0,)), ((), ())),
                          preferred_element_type=jnp.float32)
    acc_ref[pl.ds(b, 1), :] = row
    @pl.when(b == _B - 1)
    def _():
        logits_t_ref[...] = acc_ref[...].T


def _lane_gather(x, idx):
    return lax.gather(
        x, idx[:, None],
        lax.GatherDimensionNumbers(offset_dims=(), collapsed_slice_dims=(0,),
                                   start_index_map=(0,)),
        slice_sizes=(1,),
        mode=lax.GatherScatterMode.PROMISE_IN_BOUNDS)


def _bcast_lane(x, j):
    return _lane_gather(x, jnp.full((16,), j, jnp.int32))


def _route_body(logits_t_hbm, perm_out, eidx_out, loss_out, lbuf, perm_v,
                eidx_v, loss_v):
    @pl.when((lax.axis_index("c") == 0) & (lax.axis_index("s") == 0))
    def _():
        pltpu.sync_copy(logits_t_hbm, lbuf)
        lane = lax.iota(jnp.int32, 16)
        best = lbuf[0]
        bestidx = jnp.zeros((16,), jnp.int32)
        for e in range(1, _E):
            v = lbuf[e]
            gt = v > best
            best = jnp.where(gt, v, best)
            bestidx = jnp.where(gt, e, bestidx)
        counts_f = jnp.zeros((16,), jnp.float32)
        rank = jnp.zeros((16,), jnp.int32)
        for bp in range(_B):
            eb = _bcast_lane(bestidx, bp)
            counts_f = counts_f + jnp.where(lane == eb, 1.0, 0.0)
            before = (eb < bestidx) | ((eb == bestidx) & (bp < lane))
            rank = rank + jnp.where(before, 1, 0)
        perm = jnp.zeros((16,), jnp.int32)
        for bp in range(_B):
            rb = _bcast_lane(rank, bp)
            perm = jnp.where(lane == rb, bp, perm)
        eidx_sorted = _lane_gather(bestidx, perm)
        mean = _B / _E
        diff = jnp.where(lane < _E, counts_f - mean, 0.0)
        dsq = diff * diff
        tot = jnp.zeros((16,), jnp.float32)
        for e in range(_E):
            tot = tot + _bcast_lane(dsq, e)
        loss = tot * jnp.float32(2.0 / ((_E - 1) * (mean * mean + 1e-10)))
        perm_v[...] = perm
        eidx_v[...] = eidx_sorted
        loss_v[...] = loss
        pltpu.sync_copy(perm_v, perm_out)
        pltpu.sync_copy(eidx_v, eidx_out)
        pltpu.sync_copy(loss_v, loss_out)


def _dispatch_body(perm_ref, eidx_ref, p_ref, w_ref, b_ref, out_ref):
    w = w_ref[0].reshape(_PATCH, _EMB)
    y = jnp.dot(p_ref[...], w,
                preferred_element_type=jnp.float32) + b_ref[0]
    out_ref[...] = jnp.where(y == 0.0, jnp.float32(_EPS), y)


def kernel(x, router_w, router_b, expert_w, expert_b, w_gate, w_noise):
    rv = router_w.reshape(1, _PATCH).T
    w_e = expert_w.transpose(0, 2, 3, 4, 1)

    patches, logits_t = pl.pallas_call(
        _im2col_logits_body,
        grid=(_B,),
        in_specs=[
            pl.BlockSpec((1, 3 * _H, _W), lambda b: (b, 0, 0)),
            pl.BlockSpec((_PATCH, 1), lambda b: (0, 0)),
            pl.BlockSpec((1, 1), lambda b: (0, 0)),
            pl.BlockSpec((_NP, _E), lambda b: (0, 0)),
        ],
        out_specs=(
            pl.BlockSpec((_NP, _PATCH), lambda b: (b, 0)),
            pl.BlockSpec((_E, _B), lambda b: (0, 0)),
        ),
        out_shape=(
            jax.ShapeDtypeStruct((_B * _NP, _PATCH), jnp.float32),
            jax.ShapeDtypeStruct((_E, _B), jnp.float32),
        ),
        scratch_shapes=[pltpu.VMEM((_B, _E), jnp.float32)],
    )(x.reshape(_B, 3 * _H, _W), rv, router_b.reshape(1, 1), w_gate)

    route = pl.kernel(
        _route_body,
        out_type=(
            jax.ShapeDtypeStruct((_B,), jnp.int32),
            jax.ShapeDtypeStruct((_B,), jnp.int32),
            jax.ShapeDtypeStruct((_B,), jnp.float32),
        ),
        mesh=plsc.VectorSubcoreMesh(core_axis_name="c", subcore_axis_name="s"),
        scratch_types=[
            pltpu.VMEM((_E, _B), jnp.float32),
            pltpu.VMEM((_B,), jnp.int32),
            pltpu.VMEM((_B,), jnp.int32),
            pltpu.VMEM((_B,), jnp.float32),
        ],
    )
    perm, eidx_sorted, loss_v = route(logits_t)

    out = pl.pallas_call(
        _dispatch_body,
        grid_spec=pltpu.PrefetchScalarGridSpec(
            num_scalar_prefetch=2,
            grid=(_B,),
            in_specs=[
                pl.BlockSpec((_NP, _PATCH),
                             lambda r, perm_ref, eidx_ref: (perm_ref[r], 0)),
                pl.BlockSpec((1, 3, _P, _P, _EMB),
                             lambda r, perm_ref, eidx_ref:
                             (eidx_ref[r], 0, 0, 0, 0)),
                pl.BlockSpec((1, 1, _EMB),
                             lambda r, perm_ref, eidx_ref: (eidx_ref[r], 0, 0)),
            ],
            out_specs=pl.BlockSpec((_NP, _EMB),
                                   lambda r, perm_ref, eidx_ref: (perm_ref[r], 0)),
        ),
        out_shape=jax.ShapeDtypeStruct((_B * _NP, _EMB), jnp.float32),
    )(perm, eidx_sorted, patches, w_e, expert_b.reshape(_E, 1, _EMB))
    return out.reshape(_B, _NP, _EMB), loss_v[0]
